# Initial kernel scaffold; baseline (speedup 1.0000x reference)
import jax
import jax.numpy as jnp
from jax.experimental import pallas as pl


def kernel(feature, edge_index, W, b):
    raise NotImplementedError("write your pallas kernel here")


# speedup vs baseline: 4.7562x; 4.7562x over previous
import jax
import jax.numpy as jnp
from jax import lax
from jax.experimental import pallas as pl
from jax.experimental.pallas import tpu as pltpu
from jax.experimental.pallas import tpu_sc as plsc

N_NODES = 10000
N_EDGES = 320000
D_IN = 128
D_OUT = 128

NC = 2
NS = 16
NW = NC * NS
EPW = N_EDGES // NW
CHUNK = 80
NCHUNK = EPW // CHUNK
N_PAD = 10240
RPT = N_PAD // NS
NSUB = RPT // CHUNK
DEG_W = 128


def _acc_body(feat_hbm, src_hbm, dst_hbm, zfeat_hbm,
              acc_out,
              idx_s, idx_d, rows, zrows, acc_sh, sem):
    c = lax.axis_index("c")
    s = lax.axis_index("s")
    wid = s * NC + c
    base = wid * EPW
    row0 = s * RPT

    pltpu.sync_copy(zfeat_hbm, zrows)

    def zinit(k, carry):
        pltpu.sync_copy(zrows, acc_sh.at[pl.ds(row0 + k * CHUNK, CHUNK)])
        return carry

    lax.fori_loop(0, NSUB, zinit, 0)
    plsc.subcore_barrier()

    def chunk(j, carry):
        off = base + j * CHUNK
        pltpu.sync_copy(src_hbm.at[pl.ds(off, CHUNK)], idx_s)
        pltpu.sync_copy(dst_hbm.at[pl.ds(off, CHUNK)], idx_d)
        pltpu.async_copy(feat_hbm.at[idx_s], rows, sem).wait()
        pltpu.sync_copy(rows, acc_sh.at[idx_d], add=True)
        return carry

    lax.fori_loop(0, NCHUNK, chunk, 0)
    plsc.subcore_barrier()

    def wout(k, carry):
        r = row0 + k * CHUNK
        pltpu.sync_copy(acc_sh.at[pl.ds(r, CHUNK)], rows)
        pltpu.sync_copy(rows, acc_out.at[c, pl.ds(r, CHUNK)])
        return carry

    lax.fori_loop(0, NSUB, wout, 0)


def _deg_body(dst_hbm, zdeg_hbm, ones_hbm,
              deg_out,
              idx_d, ones_v, zdeg_v, deg_sh):
    c = lax.axis_index("c")
    s = lax.axis_index("s")
    wid = s * NC + c
    base = wid * EPW
    row0 = s * RPT

    pltpu.sync_copy(zdeg_hbm, zdeg_v)
    pltpu.sync_copy(ones_hbm, ones_v)

    def zinit(k, carry):
        pltpu.sync_copy(zdeg_v, deg_sh.at[pl.ds(row0 + k * CHUNK, CHUNK)])
        return carry

    lax.fori_loop(0, NSUB, zinit, 0)
    plsc.subcore_barrier()

    def chunk(j, carry):
        off = base + j * CHUNK
        pltpu.sync_copy(dst_hbm.at[pl.ds(off, CHUNK)], idx_d)
        pltpu.sync_copy(ones_v, deg_sh.at[idx_d], add=True)
        return carry

    lax.fori_loop(0, NCHUNK, chunk, 0)
    plsc.subcore_barrier()

    def wout(k, carry):
        r = row0 + k * CHUNK
        pltpu.sync_copy(deg_sh.at[pl.ds(r, CHUNK)], zdeg_v)
        pltpu.sync_copy(zdeg_v, deg_out.at[c, pl.ds(r, CHUNK)])
        return carry

    lax.fori_loop(0, NSUB, wout, 0)


def _tc_body(p_ref, g_ref, f_ref, w_ref, b_ref, o_ref):
    ssum = p_ref[0] + p_ref[1]
    deg = g_ref[0, :, 0:1] + g_ref[1, :, 0:1]
    agg = jnp.where(deg > 0.0, ssum / jnp.maximum(deg, 1.0), f_ref[...])
    h = lax.dot_general(agg, w_ref[...], (((1,), (1,)), ((), ())),
                        preferred_element_type=jnp.float32)
    o_ref[...] = jnp.maximum(h + b_ref[...], 0.0)


@jax.jit
def kernel(feature, edge_index, W, b):
    src = edge_index[0].astype(jnp.int32)
    dst = edge_index[1].astype(jnp.int32)
    zfeat = jnp.zeros((CHUNK, D_IN), jnp.float32)
    zdeg = jnp.zeros((CHUNK, DEG_W), jnp.float32)
    ones = jnp.ones((CHUNK, DEG_W), jnp.float32)

    mesh = plsc.VectorSubcoreMesh(core_axis_name="c", subcore_axis_name="s")
    acc_call = pl.kernel(
        _acc_body,
        out_type=jax.ShapeDtypeStruct((NC, N_PAD, D_IN), jnp.float32),
        mesh=mesh,
        scratch_types=[
            pltpu.VMEM((CHUNK,), jnp.int32),
            pltpu.VMEM((CHUNK,), jnp.int32),
            pltpu.VMEM((CHUNK, D_IN), jnp.float32),
            pltpu.VMEM((CHUNK, D_IN), jnp.float32),
            pltpu.VMEM_SHARED((N_PAD, D_IN), jnp.float32),
            pltpu.SemaphoreType.DMA,
        ],
    )
    partial = acc_call(feature, src, dst, zfeat)

    deg_call = pl.kernel(
        _deg_body,
        out_type=jax.ShapeDtypeStruct((NC, N_PAD, DEG_W), jnp.float32),
        mesh=mesh,
        scratch_types=[
            pltpu.VMEM((CHUNK,), jnp.int32),
            pltpu.VMEM((CHUNK, DEG_W), jnp.float32),
            pltpu.VMEM((CHUNK, DEG_W), jnp.float32),
            pltpu.VMEM_SHARED((N_PAD, DEG_W), jnp.float32),
        ],
    )
    pdeg = deg_call(dst, zdeg, ones)

    R = 1000
    out = pl.pallas_call(
        _tc_body,
        grid=(N_NODES // R,),
        in_specs=[
            pl.BlockSpec((NC, R, D_IN), lambda i: (0, i, 0)),
            pl.BlockSpec((NC, R, DEG_W), lambda i: (0, i, 0)),
            pl.BlockSpec((R, D_IN), lambda i: (i, 0)),
            pl.BlockSpec((D_OUT, D_IN), lambda i: (0, 0)),
            pl.BlockSpec((1, D_OUT), lambda i: (0, 0)),
        ],
        out_specs=pl.BlockSpec((R, D_OUT), lambda i: (i, 0)),
        out_shape=jax.ShapeDtypeStruct((N_NODES, D_OUT), jnp.float32),
    )(partial, pdeg, feature, W, b.reshape(1, D_OUT))
    return out
